# baseline (speedup 1.0000x reference)
import functools
import math

import jax
import jax.numpy as jnp
from jax import lax
from jax.experimental import pallas as pl
from jax.experimental.pallas import tpu as pltpu
from jax.experimental.pallas import tpu_sc as plsc

_NROW = 4096
_NCOL = 200
_D = 32
_NW = 32
_RPW = _NROW // _NW
_GR = 8
_NG = _RPW // _GR
_CHUNKS = ((0, 128), (128, 72))
_SCALE = math.sqrt(float(_D))

_mesh = plsc.VectorSubcoreMesh(core_axis_name="c", subcore_axis_name="s")


def _scale_group(buf):
    def body(j, carry):
        for r in range(_GR):
            for h in range(2):
                sl = pl.ds(h * 16, 16)
                buf[r, j, sl] = buf[r, j, sl] * _SCALE
        return carry

    lax.fori_loop(0, _NCOL, body, 0)


@functools.partial(
    pl.kernel,
    out_type=jax.ShapeDtypeStruct((_NROW, _NCOL, _D), jnp.float32),
    mesh=_mesh,
    compiler_params=pltpu.CompilerParams(use_tc_tiling_on_sc=False),
    scratch_types=[
        pltpu.VMEM((_RPW, _NCOL), jnp.int32),
        pltpu.VMEM((2, _GR, _NCOL, _D), jnp.float32),
        pltpu.SemaphoreType.DMA,
        pltpu.SemaphoreType.DMA,
        pltpu.SemaphoreType.DMA,
        pltpu.SemaphoreType.DMA,
    ],
)
def _emb_lookup(tokens_hbm, table_hbm, out_hbm, idx_v, rows_v,
                gsem0, gsem1, wsem0, wsem1):
    wid = lax.axis_index("s") * 2 + lax.axis_index("c")
    gsems = (gsem0, gsem1)
    wsems = (wsem0, wsem1)
    pltpu.sync_copy(tokens_hbm.at[pl.ds(wid * _RPW, _RPW)], idx_v)

    gathers = {}
    writes = {}

    def fire(g):
        b = g % 2
        descs = []
        for r in range(_GR):
            row = g * _GR + r
            for off, ln in _CHUNKS:
                descs.append(
                    pltpu.async_copy(
                        table_hbm.at[idx_v.at[row, pl.ds(off, ln)]],
                        rows_v.at[b, r, pl.ds(off, ln), :],
                        gsems[b],
                    )
                )
        gathers[g] = descs

    def flush(g):
        b = g % 2
        for d in gathers.pop(g):
            d.wait()
        _scale_group(rows_v.at[b])
        writes[g] = pltpu.async_copy(
            rows_v.at[b],
            out_hbm.at[pl.ds(wid * _RPW + g * _GR, _GR)],
            wsems[b],
        )

    fire(0)
    for g in range(_NG):
        if g + 1 < _NG:
            if g - 1 >= 0:
                writes.pop(g - 1).wait()
            fire(g + 1)
        flush(g)
    for g in sorted(writes):
        writes.pop(g).wait()


def kernel(tokens, table):
    return _emb_lookup(tokens.astype(jnp.int32), table)
